# baseline (speedup 1.0000x reference)
import functools

import jax
import jax.numpy as jnp
from jax import lax
from jax.experimental import pallas as pl
from jax.experimental.pallas import tpu as pltpu
from jax.experimental.pallas import tpu_sc as plsc

B = 4096
T = 200
D = 64
NC = 2
NS = 16
NW = NC * NS
BPW = B // NW
VP = 500000
LANES = 16
L0 = 96
L1 = T - L0
LMAX = max(L0, L1)

_mesh = plsc.VectorSubcoreMesh(core_axis_name="c", subcore_axis_name="s")


@functools.partial(
    pl.kernel,
    mesh=_mesh,
    out_type=jax.ShapeDtypeStruct((B, T, D), jnp.float32),
    scratch_types=[
        pltpu.VMEM((BPW, T), jnp.int32),
        pltpu.VMEM((T, D), jnp.float32),
        pltpu.VMEM((2, LMAX), jnp.int32),
        pltpu.VMEM((2, LMAX, 128), jnp.float32),
        pltpu.VMEM((2, LMAX, D), jnp.float32),
        pltpu.SemaphoreType.DMA,
        pltpu.SemaphoreType.DMA,
        pltpu.SemaphoreType.DMA,
        pltpu.SemaphoreType.DMA,
    ],
)
def _emb_kernel(ctx_hbm, tok_hbm, pos_hbm, out_hbm, ctx_v, pos_v, idx_v,
                rows_v, slab_v, gsem0, gsem1, ssem0, ssem1):
    gsems = (gsem0, gsem1)
    ssems = (ssem0, ssem1)
    wid = lax.axis_index("s") * NC + lax.axis_index("c")
    b0 = wid * BPW
    pltpu.sync_copy(ctx_hbm.at[pl.ds(b0, BPW)], ctx_v)
    pltpu.sync_copy(pos_hbm, pos_v)

    SLICES = (
        tuple((16 * k, 16 * k) for k in range(L0 // 16)),
        tuple((16 * k, 16 * k) for k in range(L1 // 16)) + ((L1 - 16, L1 - 16),),
    )
    T0 = (0, L0)
    LEN = (L0, L1)

    def make_indices(bl, half, q):
        t0 = T0[half]
        for src, dst in SLICES[half]:
            ids = ctx_v[bl, pl.ds(t0 + src, LANES)]
            idx_v[q, pl.ds(dst, LANES)] = lax.shift_right_logical(ids, 1)

    def start_gather(half, q):
        pltpu.async_copy(
            tok_hbm.at[idx_v.at[q, pl.ds(0, LEN[half])]],
            rows_v.at[q, pl.ds(0, LEN[half])],
            gsems[q],
        )

    def wait_gather(half, q):
        pltpu.make_async_copy(
            tok_hbm.at[pl.ds(0, LEN[half])],
            rows_v.at[q, pl.ds(0, LEN[half])],
            gsems[q],
        ).wait()

    def start_store(bl, half):
        pltpu.async_copy(
            slab_v.at[half, pl.ds(0, LEN[half])],
            out_hbm.at[b0 + bl, pl.ds(T0[half], LEN[half])],
            ssems[half],
        )

    def wait_store(half):
        pltpu.make_async_copy(
            slab_v.at[half, pl.ds(0, LEN[half])],
            out_hbm.at[0, pl.ds(0, LEN[half])],
            ssems[half],
        ).wait()

    def repack(bl, half):
        t0 = T0[half]
        for src, dst in SLICES[half]:
            hv = lax.shift_left(
                lax.bitwise_and(ctx_v[bl, pl.ds(t0 + src, LANES)], 1), 6)
            for i in range(LANES):
                off = hv[i]
                r = dst + i
                for j in range(D // LANES):
                    v = (rows_v[half, r, pl.ds(off + j * LANES, LANES)]
                         + pos_v[t0 + src + i, pl.ds(j * LANES, LANES)])
                    slab_v[half, r, pl.ds(j * LANES, LANES)] = v

    def step(bl, half, first=False, last=False):
        wait_gather(half, half)
        if not last:
            nbl, nhalf = (bl, 1) if half == 0 else (bl + 1, 0)
            make_indices(nbl, nhalf, 1 - half)
            start_gather(nhalf, 1 - half)
        if not first:
            wait_store(half)
        repack(bl, half)
        start_store(bl, half)

    make_indices(jnp.int32(0), 0, 0)
    start_gather(0, 0)
    step(jnp.int32(0), 0, first=True)
    step(jnp.int32(0), 1, first=True)

    def b_body(bl, carry):
        step(bl, 0)
        step(bl, 1)
        return carry

    lax.fori_loop(1, BPW - 1, b_body, 0)

    step(jnp.int32(BPW - 1), 0)
    step(jnp.int32(BPW - 1), 1, last=True)
    wait_store(0)
    wait_store(1)


def kernel(context, tok_table, pos_table):
    tok2 = tok_table.reshape(VP, 128)
    return _emb_kernel(context.astype(jnp.int32), tok2, pos_table)


# speedup vs baseline: 1.2340x; 1.2340x over previous
import functools

import jax
import jax.numpy as jnp
from jax import lax
from jax.experimental import pallas as pl
from jax.experimental.pallas import tpu as pltpu
from jax.experimental.pallas import tpu_sc as plsc

B = 4096
T = 200
D = 64
NC = 2
NS = 16
NW = NC * NS
BPW = B // NW
BPS = 2
NSC = BPW // BPS
SC_ROWS = BPS * T
CH0 = 128
CH1 = T - CH0
LANES = 16

_mesh = plsc.VectorSubcoreMesh(core_axis_name="c", subcore_axis_name="s")


@functools.partial(
    pl.kernel,
    mesh=_mesh,
    compiler_params=pltpu.CompilerParams(use_tc_tiling_on_sc=False),
    out_type=jax.ShapeDtypeStruct((B, T, D), jnp.float32),
    scratch_types=[
        pltpu.VMEM((BPW, T), jnp.int32),
        pltpu.VMEM((T, D), jnp.float32),
        pltpu.VMEM((2, BPS, T, D), jnp.float32),
        pltpu.SemaphoreType.DMA,
        pltpu.SemaphoreType.DMA,
        pltpu.SemaphoreType.DMA,
        pltpu.SemaphoreType.DMA,
    ],
)
def _emb_kernel(ctx_hbm, tok_hbm, pos_hbm, out_hbm, idx_v, pos_v, buf_v,
                gsem0, gsem1, ssem0, ssem1):
    gsems = (gsem0, gsem1)
    ssems = (ssem0, ssem1)
    wid = lax.axis_index("s") * NC + lax.axis_index("c")
    b0 = wid * BPW
    pltpu.sync_copy(ctx_hbm.at[pl.ds(b0, BPW)], idx_v)
    pltpu.sync_copy(pos_hbm, pos_v)

    def gather_super(s, p):
        for bb in range(BPS):
            bl = s * BPS + bb
            pltpu.async_copy(
                tok_hbm.at[idx_v.at[bl, pl.ds(0, CH0)]],
                buf_v.at[p, bb, pl.ds(0, CH0)],
                gsems[p],
            )
            pltpu.async_copy(
                tok_hbm.at[idx_v.at[bl, pl.ds(CH0, CH1)]],
                buf_v.at[p, bb, pl.ds(CH0, CH1)],
                gsems[p],
            )

    def drain_gathers(p):
        pltpu.make_async_copy(
            out_hbm.at[pl.ds(0, BPS)],
            buf_v.at[p],
            gsems[p],
        ).wait()

    def wait_store(p):
        pltpu.make_async_copy(
            buf_v.at[p],
            out_hbm.at[pl.ds(0, BPS)],
            ssems[p],
        ).wait()

    def add_pos(p):
        for bb in range(BPS):

            def row_body(i, carry, bb=bb):
                for j in range(D // LANES):
                    v = pos_v[i, pl.ds(j * LANES, LANES)]
                    plsc.addupdate(
                        buf_v.at[p, bb, i, pl.ds(j * LANES, LANES)], v
                    )
                return carry

            lax.fori_loop(0, T, row_body, 0, unroll=4)

    def start_store(s, p):
        pltpu.async_copy(
            buf_v.at[p],
            out_hbm.at[pl.ds(b0 + s * BPS, BPS)],
            ssems[p],
        )

    def step(s, p, first=False, last=False):
        q = 1 - p
        drain_gathers(p)
        if not last:
            if not first:
                wait_store(q)
            gather_super(s + 1, q)
        add_pos(p)
        start_store(s, p)

    gather_super(0, 0)
    step(0, 0, first=True)
    step(jnp.int32(1), 1)

    def pair_body(g, carry):
        s = g * 2
        step(s, 0)
        step(s + 1, 1)
        return carry

    lax.fori_loop(1, NSC // 2 - 1, pair_body, 0)

    step(jnp.int32(NSC - 2), 0)
    step(jnp.int32(NSC - 1), 1, last=True)
    wait_store(0)
    wait_store(1)


def kernel(context, tok_table, pos_table):
    return _emb_kernel(context.astype(jnp.int32), tok_table, pos_table)
